# Initial kernel scaffold; baseline (speedup 1.0000x reference)
import jax
import jax.numpy as jnp
from jax.experimental import pallas as pl


def kernel(h, code_emb, code_anchor, feature_importance, importance_temperature, W1, b1, W2, b2):
    raise NotImplementedError("write your pallas kernel here")


# speedup vs baseline: 4.4455x; 4.4455x over previous
import functools
import math

import jax
import jax.numpy as jnp
from jax.experimental import pallas as pl

B, T = 2, 2048
D = 1024
CD = 256
E = 8
TOPK = 2
HID2 = 2 * D
SLICE = 1024 // E
N = B * T


def _router_body(ce_ref, anchor_ref, g_ref, fi_ref, temp_ref,
                 ew_ref, imp_ref, aux_ref):
    ce = ce_ref[...]
    anchor = anchor_ref[...]
    an = anchor / jnp.maximum(
        jnp.sqrt(jnp.sum(anchor * anchor, axis=1, keepdims=True)), 1e-12)
    cn = ce / jnp.maximum(
        jnp.sqrt(jnp.sum(ce * ce, axis=1, keepdims=True)), 1e-12)
    logits = jnp.dot(cn, an.T, preferred_element_type=jnp.float32) * 0.125
    z = (logits + g_ref[...]) / 0.1
    z = z - jnp.max(z, axis=1, keepdims=True)
    ez = jnp.exp(z)
    y = ez / jnp.sum(ez, axis=1, keepdims=True)
    eidx = jax.lax.broadcasted_iota(jnp.int32, (N, E), 1)
    m1 = jnp.max(y, axis=1, keepdims=True)
    i1 = jnp.min(jnp.where(y == m1, eidx, E), axis=1, keepdims=True)
    sel1 = eidx == i1
    y2 = jnp.where(sel1, -jnp.inf, y)
    m2 = jnp.max(y2, axis=1, keepdims=True)
    i2 = jnp.min(jnp.where(y2 == m2, eidx, E), axis=1, keepdims=True)
    sel = sel1 | (eidx == i2)
    ew = jnp.where(sel, y, 0.0)
    ew_ref[...] = ew
    counts = ew[:T, :] + ew[T:, :]
    nelem = T * E
    mean = jnp.sum(counts) / nelem
    var = jnp.sum((counts - mean) ** 2) / (nelem - 1)
    std = jnp.sqrt(var)
    load = counts / (jnp.sum(counts) + 1e-8)
    load_ent = -jnp.sum(load * jnp.log(load + 1e-8))
    routing_loss = 0.5 * (std + load_ent)
    tclip = jnp.clip(temp_ref[0, 0], 0.1, 5.0)
    fi = fi_ref[...] / tclip
    fi = fi - jnp.max(fi, axis=1, keepdims=True)
    efi = jnp.exp(fi)
    imp = efi / jnp.sum(efi, axis=1, keepdims=True)
    imp_ref[...] = imp
    ent = -jnp.sum(imp * jnp.log(imp + 1e-8)) / E
    aux_ref[...] = jnp.broadcast_to(routing_loss - 0.01 * ent, (1, 1))


def _mlp_body(imp_ref, h_ref, ce_ref, w1_ref, b1_ref, w2_ref, b2_ref, ewt_ref,
              out_ref):
    xh = h_ref[...] * imp_ref[0]
    a = jnp.dot(xh, w1_ref[0, :D, :], preferred_element_type=jnp.float32)
    a += jnp.dot(ce_ref[...], w1_ref[0, D:, :],
                 preferred_element_type=jnp.float32)
    a += b1_ref[0]
    hdn = 0.5 * a * (1.0 + jax.lax.erf(a * (1.0 / math.sqrt(2.0))))
    out = jnp.dot(hdn, w2_ref[0], preferred_element_type=jnp.float32)
    out += b2_ref[0]
    tt = ewt_ref.shape[-1]
    scale = ewt_ref[0, 0, 0].reshape(tt, 1)
    out_ref[...] = out * scale


def kernel(h, code_emb, code_anchor, feature_importance,
           importance_temperature, W1, b1, W2, b2):
    h2 = h.reshape(N, D)
    ce2 = code_emb.reshape(N, CD)
    g = jax.random.gumbel(jax.random.key(42), (N, E), dtype=jnp.float32)
    temp = importance_temperature.reshape(1, 1)

    ew, imp, aux = pl.pallas_call(
        _router_body,
        out_shape=[
            jax.ShapeDtypeStruct((N, E), jnp.float32),
            jax.ShapeDtypeStruct((E, D), jnp.float32),
            jax.ShapeDtypeStruct((1, 1), jnp.float32),
        ],
    )(ce2, code_anchor, g, feature_importance, temp)

    TT = 512
    NT = N // TT
    ewt = ew.T.reshape(E, NT, 1, TT)

    out = pl.pallas_call(
        _mlp_body,
        grid=(E, NT),
        in_specs=[
            pl.BlockSpec((1, 1, D), lambda e, t: (e, 0, 0)),
            pl.BlockSpec((TT, D), lambda e, t: (t, 0)),
            pl.BlockSpec((TT, CD), lambda e, t: (t, 0)),
            pl.BlockSpec((1, D + CD, HID2), lambda e, t: (e, 0, 0)),
            pl.BlockSpec((1, 1, HID2), lambda e, t: (e, 0, 0)),
            pl.BlockSpec((1, HID2, SLICE), lambda e, t: (e, 0, 0)),
            pl.BlockSpec((1, 1, SLICE), lambda e, t: (e, 0, 0)),
            pl.BlockSpec((1, 1, 1, TT), lambda e, t: (e, t, 0, 0)),
        ],
        out_specs=pl.BlockSpec((TT, SLICE), lambda e, t: (t, e)),
        out_shape=jax.ShapeDtypeStruct((N, E * SLICE), jnp.float32),
    )(imp.reshape(E, 1, D), h2, ce2, W1, b1.reshape(E, 1, HID2), W2,
      b2.reshape(E, 1, SLICE), ewt)

    return out.reshape(B, T, E * SLICE), aux.reshape(())
